# speedup vs baseline: 1.5593x; 1.5593x over previous
import jax
import jax.numpy as jnp
from jax import lax
from jax.experimental import pallas as pl

DIM = 128
NUM_MODELS = 64
TEXT_DIM = 1536
BATCH = 4096


def _tc_body(ids_ref, prompt_ref, p_ref, wt_ref, wcls_ref, out_ref):
    pe = lax.dot_general(
        prompt_ref[...], wt_ref[...],
        dimension_numbers=(((1,), (1,)), ((), ())),
        preferred_element_type=jnp.float32,
    )
    w = pe * wcls_ref[...]
    p = p_ref[...]
    norm = jnp.sqrt(jnp.sum(p * p, axis=1, keepdims=True))
    dots = lax.dot_general(
        p, w, dimension_numbers=(((1,), (1,)), ((), ())),
        preferred_element_type=jnp.float32,
    )
    s = dots / jnp.maximum(norm, 1e-12)
    ids = ids_ref[...]
    iota = lax.broadcasted_iota(jnp.int32, (BATCH, NUM_MODELS), 1)
    onehot = (ids == iota).astype(jnp.float32)
    out_ref[...] = lax.dot_general(
        onehot, s, dimension_numbers=(((1,), (0,)), ((), ())),
        preferred_element_type=jnp.float32,
    )


def kernel(model_id, prompt_embed, P, W_text, W_cls):
    ids = model_id.astype(jnp.int32).reshape(BATCH, 1)
    prompt = prompt_embed.reshape(1, TEXT_DIM)
    out = pl.pallas_call(
        _tc_body,
        out_shape=jax.ShapeDtypeStruct((BATCH, 1), jnp.float32),
    )(ids, prompt, P, W_text, W_cls)
    return out.reshape(BATCH)
